# baseline (speedup 1.0000x reference)
import functools

import jax
import jax.numpy as jnp
from jax import lax
from jax.experimental import pallas as pl
from jax.experimental.pallas import tpu as pltpu
from jax.experimental.pallas import tpu_sc as plsc

EMBED_DIM = 64
NUM_NEG = 200
N_ROWS = 208
N_GROUPS = N_ROWS // 16
VALID_LAST = 9


def _softplus(z):
    e = jnp.exp(-jnp.abs(z))
    t = e / (e + 2.0)
    t2 = t * t
    ln_y = 2.0 * t * (1.0 + t2 * (1.0 / 3.0 + t2 * (1.0 / 5.0 + t2 * (1.0 / 7.0 + t2 * (1.0 / 9.0)))))
    return jnp.maximum(z, 0.0) + ln_y


def _sc_kernel(input_hbm, output_hbm, idx_c_hbm, idx_pn_hbm, off_c_hbm, off_pn_hbm,
               out_hbm, idx_c_v, idx_pn_v, off_c_v, off_pn_v, rows_c_v, rows_v,
               out_v, sem):
    is_lead = jnp.logical_and(lax.axis_index("c") == 0, lax.axis_index("s") == 0)

    @pl.when(is_lead)
    def _():
        pltpu.sync_copy(idx_c_hbm, idx_c_v)
        pltpu.sync_copy(idx_pn_hbm, idx_pn_v)
        pltpu.sync_copy(off_c_hbm, off_c_v)
        pltpu.sync_copy(off_pn_hbm, off_pn_v)

        cp0 = pltpu.make_async_copy(input_hbm.at[idx_c_v], rows_c_v, sem)
        cp1 = pltpu.make_async_copy(
            output_hbm.at[idx_pn_v.at[pl.ds(0, 104)]], rows_v.at[pl.ds(0, 104)], sem)
        cp2 = pltpu.make_async_copy(
            output_hbm.at[idx_pn_v.at[pl.ds(104, 104)]], rows_v.at[pl.ds(104, 104)], sem)
        cp0.start()
        cp1.start()
        cp2.start()
        cp0.wait()
        cp1.wait()
        cp2.wait()

        lanes = lax.iota(jnp.int32, 16)
        row_ids = [lanes + (16 * g) for g in range(N_GROUPS)]
        zero_ids = jnp.zeros((16,), jnp.int32)
        off_c = off_c_v[...]
        offs = [off_pn_v[pl.ds(16 * g, 16)] for g in range(N_GROUPS)]

        def body(j, accs):
            cej = plsc.load_gather(rows_c_v, [zero_ids, off_c + j])
            return tuple(
                acc + cej * plsc.load_gather(rows_v, [row_ids[g], offs[g] + j])
                for g, acc in enumerate(accs)
            )

        zero = jnp.zeros((16,), jnp.float32)
        accs = lax.fori_loop(0, EMBED_DIM, body, (zero,) * N_GROUPS)

        total = zero
        for g in range(N_GROUPS):
            d = accs[g]
            if g == 0:
                d = jnp.where(lanes == 0, -d, d)
            if g == N_GROUPS - 1:
                d = jnp.where(lanes < VALID_LAST, d, -1e30)
            total = total + _softplus(d)

        out_v[...] = jnp.full((16,), jnp.sum(total))
        pltpu.sync_copy(out_v, out_hbm)


@jax.jit
def _run(center_word, positive_words, negative_words, input_embeddings, output_embeddings):
    vocab = input_embeddings.shape[0]
    inp2 = input_embeddings.reshape(vocab // 2, 2 * EMBED_DIM)
    out2 = output_embeddings.reshape(vocab // 2, 2 * EMBED_DIM)
    c = center_word.astype(jnp.int32)
    pn = jnp.concatenate([
        positive_words.astype(jnp.int32),
        negative_words.astype(jnp.int32),
        jnp.zeros((N_ROWS - 1 - NUM_NEG,), jnp.int32),
    ])
    idx_c = jnp.broadcast_to(c >> 1, (16,))
    off_c = jnp.broadcast_to((c & 1) * EMBED_DIM, (16,))
    idx_pn = pn >> 1
    off_pn = (pn & 1) * EMBED_DIM
    mesh = plsc.VectorSubcoreMesh(core_axis_name="c", subcore_axis_name="s")
    k = functools.partial(
        pl.kernel,
        mesh=mesh,
        compiler_params=pltpu.CompilerParams(
            use_tc_tiling_on_sc=False, needs_layout_passes=False),
        out_type=jax.ShapeDtypeStruct((16,), jnp.float32),
        scratch_types=[
            pltpu.VMEM((16,), jnp.int32),
            pltpu.VMEM((N_ROWS,), jnp.int32),
            pltpu.VMEM((16,), jnp.int32),
            pltpu.VMEM((N_ROWS,), jnp.int32),
            pltpu.VMEM((16, 2 * EMBED_DIM), jnp.float32),
            pltpu.VMEM((N_ROWS, 2 * EMBED_DIM), jnp.float32),
            pltpu.VMEM((16,), jnp.float32),
            pltpu.SemaphoreType.DMA,
        ],
    )(_sc_kernel)
    res = k(inp2, out2, idx_c, idx_pn, off_c, off_pn)
    return res[0].reshape(1, 1)


def kernel(center_word, positive_words, negative_words, input_embeddings, output_embeddings):
    return _run(center_word, positive_words, negative_words,
                input_embeddings, output_embeddings)


# speedup vs baseline: 1.0015x; 1.0015x over previous
import functools

import jax
import jax.numpy as jnp
from jax import lax
from jax.experimental import pallas as pl
from jax.experimental.pallas import tpu as pltpu
from jax.experimental.pallas import tpu_sc as plsc

EMBED_DIM = 64
NUM_NEG = 200
N_ROWS = 208
N_GROUPS = N_ROWS // 16
VALID_LAST = 9


def _softplus(z):
    e = jnp.exp(-jnp.abs(z))
    t = e / (e + 2.0)
    t2 = t * t
    ln_y = 2.0 * t * (1.0 + t2 * (1.0 / 3.0 + t2 * (1.0 / 5.0 + t2 * (1.0 / 7.0 + t2 * (1.0 / 9.0)))))
    return jnp.maximum(z, 0.0) + ln_y


def _sc_kernel(input_hbm, output_hbm, idx_c_hbm, idx_pn_hbm, off_c_hbm, off_pn_hbm,
               out_hbm, idx_c_v, idx_pn_v, off_c_v, off_pn_v, rows_c_v, rows_v,
               out_v, sem):
    is_lead = jnp.logical_and(lax.axis_index("c") == 0, lax.axis_index("s") == 0)

    @pl.when(is_lead)
    def _():
        pltpu.sync_copy(idx_c_hbm, idx_c_v)
        pltpu.sync_copy(idx_pn_hbm, idx_pn_v)
        pltpu.sync_copy(off_c_hbm, off_c_v)
        pltpu.sync_copy(off_pn_hbm, off_pn_v)

        cp0 = pltpu.make_async_copy(input_hbm.at[idx_c_v], rows_c_v, sem)
        cp1 = pltpu.make_async_copy(
            output_hbm.at[idx_pn_v.at[pl.ds(0, 104)]], rows_v.at[pl.ds(0, 104)], sem)
        cp2 = pltpu.make_async_copy(
            output_hbm.at[idx_pn_v.at[pl.ds(104, 104)]], rows_v.at[pl.ds(104, 104)], sem)
        cp0.start()
        cp1.start()
        cp2.start()
        cp0.wait()
        cp1.wait()
        cp2.wait()

        lanes = lax.iota(jnp.int32, 16)
        row_ids = [lanes + (16 * g) for g in range(N_GROUPS)]
        zero_ids = jnp.zeros((16,), jnp.int32)
        off_c = off_c_v[...]
        offs = [off_pn_v[pl.ds(16 * g, 16)] for g in range(N_GROUPS)]

        def body(j, accs):
            cej = plsc.load_gather(rows_c_v, [zero_ids, off_c + j])
            return tuple(
                acc + cej * plsc.load_gather(rows_v, [row_ids[g], offs[g] + j])
                for g, acc in enumerate(accs)
            )

        zero = jnp.zeros((16,), jnp.float32)
        accs = lax.fori_loop(0, EMBED_DIM, body, (zero,) * N_GROUPS)

        total = zero
        for g in range(N_GROUPS):
            d = accs[g]
            if g == 0:
                d = jnp.where(lanes == 0, -d, d)
            if g == N_GROUPS - 1:
                d = jnp.where(lanes < VALID_LAST, d, -1e30)
            total = total + _softplus(d)

        out_v[...] = jnp.full((16,), jnp.sum(total))
        pltpu.sync_copy(out_v, out_hbm)


@jax.jit
def _run(center_word, positive_words, negative_words, input_embeddings, output_embeddings):
    vocab = input_embeddings.shape[0]
    inp2 = input_embeddings.reshape(vocab // 2, 2 * EMBED_DIM)
    out2 = output_embeddings.reshape(vocab // 2, 2 * EMBED_DIM)
    c = center_word.astype(jnp.int32)
    pn = jnp.concatenate([
        positive_words.astype(jnp.int32),
        negative_words.astype(jnp.int32),
        jnp.zeros((N_ROWS - 1 - NUM_NEG,), jnp.int32),
    ])
    idx_c = jnp.broadcast_to(c >> 1, (16,))
    off_c = jnp.broadcast_to((c & 1) * EMBED_DIM, (16,))
    idx_pn = pn >> 1
    off_pn = (pn & 1) * EMBED_DIM
    mesh = plsc.VectorSubcoreMesh(core_axis_name="c", subcore_axis_name="s")
    k = functools.partial(
        pl.kernel,
        mesh=mesh,
        compiler_params=pltpu.CompilerParams(needs_layout_passes=False),
        out_type=jax.ShapeDtypeStruct((16,), jnp.float32),
        scratch_types=[
            pltpu.VMEM((16,), jnp.int32),
            pltpu.VMEM((N_ROWS,), jnp.int32),
            pltpu.VMEM((16,), jnp.int32),
            pltpu.VMEM((N_ROWS,), jnp.int32),
            pltpu.VMEM((16, 2 * EMBED_DIM), jnp.float32),
            pltpu.VMEM((N_ROWS, 2 * EMBED_DIM), jnp.float32),
            pltpu.VMEM((16,), jnp.float32),
            pltpu.SemaphoreType.DMA,
        ],
    )(_sc_kernel)
    res = k(inp2, out2, idx_c, idx_pn, off_c, off_pn)
    return res[0].reshape(1, 1)


def kernel(center_word, positive_words, negative_words, input_embeddings, output_embeddings):
    return _run(center_word, positive_words, negative_words,
                input_embeddings, output_embeddings)


# speedup vs baseline: 40.5740x; 40.5138x over previous
import functools

import jax
import jax.numpy as jnp
from jax import lax
from jax.experimental import pallas as pl
from jax.experimental.pallas import tpu as pltpu
from jax.experimental.pallas import tpu_sc as plsc

EMBED_DIM = 64
NUM_NEG = 200
N_TARGETS = 1 + NUM_NEG
PER_TILE = 13
IDX_PAD = 240


def _softplus(z):
    e = jnp.exp(-jnp.abs(z))
    t = e / (e + 2.0)
    t2 = t * t
    ln_y = 2.0 * t * (1.0 + t2 * (1.0 / 3.0 + t2 * (1.0 / 5.0 + t2 * (1.0 / 7.0 + t2 * (1.0 / 9.0)))))
    return jnp.maximum(z, 0.0) + ln_y


def _sc_kernel(inp_t_hbm, out_t_hbm, cen_hbm, idx_hbm, out_hbm,
               idx_v, cen_v, ce_buf, win_buf, sp_v, sp_shared, red_buf,
               out_v, sem):
    on_core0 = lax.axis_index("c") == 0

    @pl.when(on_core0)
    def _():
        w = lax.axis_index("s")

        pltpu.sync_copy(idx_hbm, idx_v)
        pltpu.sync_copy(cen_hbm, cen_v)

        c = cen_v[...][0]
        cp_ce = pltpu.make_async_copy(
            inp_t_hbm.at[:, pl.ds(pl.multiple_of((c >> 7) << 7, 128), 128)],
            ce_buf, sem)
        cp_ce.start()
        cps = []
        for i in range(PER_TILE):
            r = idx_v[pl.ds(w * PER_TILE + i, 16)][0]
            cp = pltpu.make_async_copy(
                out_t_hbm.at[:, pl.ds(pl.multiple_of((r >> 7) << 7, 128), 128)],
                win_buf.at[i], sem)
            cp.start()
            cps.append(cp)
        cp_ce.wait()
        for cp in cps:
            cp.wait()

        lanes = lax.iota(jnp.int32, 16)
        idx_vec = idx_v[pl.ds(w * PER_TILE, 16)]
        off_vec = jnp.bitwise_and(idx_vec, 127)
        i_vec = jnp.where(lanes < PER_TILE, lanes, 0)
        c_off = jnp.full((16,), jnp.bitwise_and(c, 127), jnp.int32)

        def body(j, acc):
            jv = jnp.full((16,), j, jnp.int32)
            cej = plsc.load_gather(ce_buf, [jv, c_off])
            col = plsc.load_gather(win_buf, [i_vec, jv, off_vec])
            return acc + cej * col

        acc = lax.fori_loop(0, EMBED_DIM, body, jnp.zeros((16,), jnp.float32),
                            unroll=8)

        t_vec = lanes + w * PER_TILE
        d = jnp.where(t_vec == 0, -acc, acc)
        valid = jnp.logical_and(lanes < PER_TILE, t_vec < N_TARGETS)
        d = jnp.where(valid, d, -1e30)
        sp_v[...] = _softplus(d)
        pltpu.sync_copy(sp_v, sp_shared.at[w])

    plsc.subcore_barrier()

    @pl.when(jnp.logical_and(on_core0, lax.axis_index("s") == 0))
    def _():
        pltpu.sync_copy(sp_shared, red_buf)
        total = red_buf[0, pl.ds(0, 16)]
        for ww in range(1, 16):
            total = total + red_buf[ww, pl.ds(0, 16)]
        out_v[...] = jnp.full((16,), jnp.sum(total))
        pltpu.sync_copy(out_v, out_hbm)


@jax.jit
def _run(center_word, positive_words, negative_words, input_embeddings, output_embeddings):
    inp_t = input_embeddings.T
    out_t = output_embeddings.T
    cen = jnp.broadcast_to(center_word.astype(jnp.int32), (16,))
    idx = jnp.concatenate([
        positive_words.astype(jnp.int32),
        negative_words.astype(jnp.int32),
        jnp.zeros((IDX_PAD - N_TARGETS,), jnp.int32),
    ])
    mesh = plsc.VectorSubcoreMesh(core_axis_name="c", subcore_axis_name="s")
    k = functools.partial(
        pl.kernel,
        mesh=mesh,
        compiler_params=pltpu.CompilerParams(needs_layout_passes=False),
        out_type=jax.ShapeDtypeStruct((16,), jnp.float32),
        scratch_types=[
            pltpu.VMEM((IDX_PAD,), jnp.int32),
            pltpu.VMEM((16,), jnp.int32),
            pltpu.VMEM((EMBED_DIM, 128), jnp.float32),
            pltpu.VMEM((PER_TILE, EMBED_DIM, 128), jnp.float32),
            pltpu.VMEM((16,), jnp.float32),
            pltpu.VMEM_SHARED((16, 16), jnp.float32),
            pltpu.VMEM((16, 16), jnp.float32),
            pltpu.VMEM((16,), jnp.float32),
            pltpu.SemaphoreType.DMA,
        ],
    )(_sc_kernel)
    res = k(inp_t, out_t, cen, idx)
    return res[0].reshape(1, 1)


def kernel(center_word, positive_words, negative_words, input_embeddings, output_embeddings):
    return _run(center_word, positive_words, negative_words,
                input_embeddings, output_embeddings)
